# Initial kernel scaffold; baseline (speedup 1.0000x reference)
import jax
import jax.numpy as jnp
from jax.experimental import pallas as pl


def kernel(x, edge_index, batch, part_ids, embeddings, W_enc, b_enc, W_g0, b_g0, W_g1, b_g1, W_g2, b_g2, W_a1, b_a1, W_a2, b_a2, W_v1, b_v1, W_v2, b_v2):
    raise NotImplementedError("write your pallas kernel here")


# speedup vs baseline: 2.1614x; 2.1614x over previous
import functools

import jax
import jax.numpy as jnp
from jax.experimental import pallas as pl
from jax.experimental.pallas import tpu as pltpu

N = 10000
D = 256
H = 512
PE = 64
G = 16


def _enc_body(x_ref, w_ref, pemb_ref, b_ref, o_ref):
    acc = jnp.dot(x_ref[...], w_ref[...], preferred_element_type=jnp.float32)
    o_ref[...] = acc + pemb_ref[...] + b_ref[...]


def _encoder(x, w, pemb, b):
    blk = 1000
    grid = (N // blk,)
    return pl.pallas_call(
        _enc_body,
        grid=grid,
        in_specs=[
            pl.BlockSpec((blk, D), lambda i: (i, 0)),
            pl.BlockSpec((D, H), lambda i: (0, 0)),
            pl.BlockSpec((blk, H), lambda i: (i, 0)),
            pl.BlockSpec((1, H), lambda i: (0, 0)),
        ],
        out_specs=pl.BlockSpec((blk, H), lambda i: (i, 0)),
        out_shape=jax.ShapeDtypeStruct((N, H), jnp.float32),
    )(x, w, pemb, b)


def kernel(x, edge_index, batch, part_ids, embeddings, W_enc, b_enc,
           W_g0, b_g0, W_g1, b_g1, W_g2, b_g2,
           W_a1, b_a1, W_a2, b_a2, W_v1, b_v1, W_v2, b_v2):
    n = x.shape[0]
    loops = jnp.arange(n, dtype=edge_index.dtype)
    src0, dst0 = edge_index[0], edge_index[1]

    table = embeddings @ W_enc[:PE]
    pemb_h = jnp.take(table, part_ids, axis=0)
    h = _encoder(x, W_enc[PE:], pemb_h, b_enc[None, :])

    deg = jax.ops.segment_sum(jnp.ones_like(src0, dtype=h.dtype), dst0,
                              num_segments=n) + 1.0
    dinv = 1.0 / jnp.sqrt(deg)

    for W, b in ((W_g0, b_g0), (W_g1, b_g1), (W_g2, b_g2)):
        ms = (h @ W) * dinv[:, None]
        agg = jax.ops.segment_sum(ms[src0], dst0, num_segments=n) + ms
        h = jax.nn.relu(agg * dinv[:, None] + b)

    adv = jax.nn.relu(h @ W_a1 + b_a1) @ W_a2 + b_a2
    cnt = jnp.maximum(jax.ops.segment_sum(jnp.ones((n,), h.dtype), batch,
                                          num_segments=G), 1.0)
    adv_mean = (jax.ops.segment_sum(adv, batch, num_segments=G) / cnt[:, None])[batch]
    value_x = jax.ops.segment_sum(h, batch, num_segments=G) / cnt[:, None]
    value = (jax.nn.relu(value_x @ W_v1 + b_v1) @ W_v2 + b_v2)[batch]
    return value + adv - adv_mean


# speedup vs baseline: 6.5827x; 3.0456x over previous
import functools

import jax
import jax.numpy as jnp
from jax import lax
from jax.experimental import pallas as pl
from jax.experimental.pallas import tpu as pltpu
from jax.experimental.pallas import tpu_sc as plsc

N = 10000
E = 160000
D = 256
H = 512
PE = 64
G = 16

BLK = 1000
NCH = 4
CW = 128
NT = 16
EC = 80
ECH = E // NT // EC
RPT = 624
RTAIL = N - NT * RPT


def _enc_body(pe_ref, x_ref, w_ref, be_ref, o_ref):
    lhs = jnp.concatenate([pe_ref[...], x_ref[...]], axis=1)
    o_ref[...] = (jnp.dot(lhs, w_ref[...], preferred_element_type=jnp.float32)
                  + be_ref[...])


def _encoder(pe, x, w, be):
    return pl.pallas_call(
        _enc_body,
        grid=(N // BLK,),
        in_specs=[
            pl.BlockSpec((BLK, PE), lambda i: (i, 0)),
            pl.BlockSpec((BLK, D), lambda i: (i, 0)),
            pl.BlockSpec((PE + D, H), lambda i: (0, 0)),
            pl.BlockSpec((1, H), lambda i: (0, 0)),
        ],
        out_specs=pl.BlockSpec((BLK, H), lambda i: (i, 0)),
        out_shape=jax.ShapeDtypeStruct((N, H), jnp.float32),
    )(pe, x, w, be)


def _mm0_body(h_ref, w_ref, deg_ref, o0, o1, o2, o3):
    dinv = 1.0 / jnp.sqrt(deg_ref[...])
    res = jnp.dot(h_ref[...], w_ref[...],
                  preferred_element_type=jnp.float32) * dinv
    for k, o in enumerate((o0, o1, o2, o3)):
        o[...] = res[:, k * CW:(k + 1) * CW]


def _layer_mm0(h, w, deg):
    return pl.pallas_call(
        _mm0_body,
        grid=(N // BLK,),
        in_specs=[
            pl.BlockSpec((BLK, H), lambda i: (i, 0)),
            pl.BlockSpec((H, H), lambda i: (0, 0)),
            pl.BlockSpec((BLK, 1), lambda i: (i, 0)),
        ],
        out_specs=[pl.BlockSpec((BLK, CW), lambda i: (i, 0))] * NCH,
        out_shape=[jax.ShapeDtypeStruct((N, CW), jnp.float32)] * NCH,
    )(h, w, deg)


def _mm_body(a0, a1, a2, a3, w_ref, deg_ref, bp_ref, o0, o1, o2, o3):
    dinv = 1.0 / jnp.sqrt(deg_ref[...])
    agg = jnp.concatenate([a0[...], a1[...], a2[...], a3[...]], axis=1)
    x = jnp.maximum(agg * dinv + bp_ref[...], 0.0)
    res = jnp.dot(x, w_ref[...], preferred_element_type=jnp.float32) * dinv
    for k, o in enumerate((o0, o1, o2, o3)):
        o[...] = res[:, k * CW:(k + 1) * CW]


def _layer_mm(aggs, w, deg, b_prev):
    return pl.pallas_call(
        _mm_body,
        grid=(N // BLK,),
        in_specs=[pl.BlockSpec((BLK, CW), lambda i: (i, 0))] * NCH + [
            pl.BlockSpec((H, H), lambda i: (0, 0)),
            pl.BlockSpec((BLK, 1), lambda i: (i, 0)),
            pl.BlockSpec((1, H), lambda i: (0, 0)),
        ],
        out_specs=[pl.BlockSpec((BLK, CW), lambda i: (i, 0))] * NCH,
        out_shape=[jax.ShapeDtypeStruct((N, CW), jnp.float32)] * NCH,
    )(*aggs, w, deg, b_prev)


def _sc_pass(ms_hbm, out_hbm, acc, srcl, dstl, buf, sem, s):
    pltpu.sync_copy(ms_hbm.at[pl.ds(s * RPT, RPT)], acc.at[pl.ds(s * RPT, RPT)])

    @pl.when(s == 0)
    def _():
        pltpu.sync_copy(ms_hbm.at[pl.ds(NT * RPT, RTAIL)],
                        acc.at[pl.ds(NT * RPT, RTAIL)])

    plsc.subcore_barrier()

    def chunk(j, carry):
        pltpu.async_copy(ms_hbm.at[srcl.at[j]], buf, sem).wait()
        pltpu.sync_copy(buf, acc.at[dstl.at[j]], add=True)
        return carry

    lax.fori_loop(0, ECH, chunk, 0)
    plsc.subcore_barrier()
    pltpu.sync_copy(acc.at[pl.ds(s * RPT, RPT)], out_hbm.at[pl.ds(s * RPT, RPT)])

    @pl.when(s == 0)
    def _():
        pltpu.sync_copy(acc.at[pl.ds(NT * RPT, RTAIL)],
                        out_hbm.at[pl.ds(NT * RPT, RTAIL)])

    plsc.subcore_barrier()


def _scatter_body(ms0, ms1, ms2, ms3, src_hbm, dst_hbm,
                  out0, out1, out2, out3, acc, srcl, dstl, buf, sem):
    c = lax.axis_index("c")
    s = lax.axis_index("s")
    pltpu.sync_copy(src_hbm.at[s], srcl)
    pltpu.sync_copy(dst_hbm.at[s], dstl)
    mss = (ms0, ms1, ms2, ms3)
    outs = (out0, out1, out2, out3)
    for half in range(2):
        @pl.when(c == 0)
        def _():
            _sc_pass(mss[half], outs[half], acc, srcl, dstl, buf, sem, s)

        @pl.when(c == 1)
        def _():
            _sc_pass(mss[2 + half], outs[2 + half], acc, srcl, dstl, buf, sem, s)


def _make_scatter():
    mesh = plsc.VectorSubcoreMesh(core_axis_name="c", subcore_axis_name="s")
    return pl.kernel(
        _scatter_body,
        out_type=[jax.ShapeDtypeStruct((N, CW), jnp.float32)] * NCH,
        mesh=mesh,
        scratch_types=[
            pltpu.VMEM_SHARED((N, CW), jnp.float32),
            pltpu.VMEM((ECH, EC), jnp.int32),
            pltpu.VMEM((ECH, EC), jnp.int32),
            pltpu.VMEM((EC, CW), jnp.float32),
            pltpu.SemaphoreType.DMA,
        ],
    )


def kernel(x, edge_index, batch, part_ids, embeddings, W_enc, b_enc,
           W_g0, b_g0, W_g1, b_g1, W_g2, b_g2,
           W_a1, b_a1, W_a2, b_a2, W_v1, b_v1, W_v2, b_v2):
    n = x.shape[0]
    src0, dst0 = edge_index[0], edge_index[1]
    src_r = src0.reshape(NT, ECH, EC)
    dst_r = dst0.reshape(NT, ECH, EC)

    deg = jax.ops.segment_sum(jnp.ones_like(src0, dtype=jnp.float32), dst0,
                              num_segments=n) + 1.0
    deg2 = deg[:, None]

    pe = jnp.take(embeddings, part_ids, axis=0)
    h = _encoder(pe, x, W_enc, b_enc[None, :])

    scat = _make_scatter()
    ms = _layer_mm0(h, W_g0, deg2)
    aggs = scat(*ms, src_r, dst_r)
    for W, b_prev in ((W_g1, b_g0), (W_g2, b_g1)):
        ms = _layer_mm(aggs, W, deg2, b_prev[None, :])
        aggs = scat(*ms, src_r, dst_r)

    dinv = 1.0 / jnp.sqrt(deg)
    agg = jnp.concatenate(aggs, axis=1)
    h = jax.nn.relu(agg * dinv[:, None] + b_g2)

    adv = jax.nn.relu(h @ W_a1 + b_a1) @ W_a2 + b_a2
    cnt = jnp.maximum(jax.ops.segment_sum(jnp.ones((n,), h.dtype), batch,
                                          num_segments=G), 1.0)
    adv_mean = (jax.ops.segment_sum(adv, batch, num_segments=G) / cnt[:, None])[batch]
    value_x = jax.ops.segment_sum(h, batch, num_segments=G) / cnt[:, None]
    value = (jax.nn.relu(value_x @ W_v1 + b_v1) @ W_v2 + b_v2)[batch]
    return value + adv - adv_mean
